# baseline (speedup 1.0000x reference)
import dataclasses
import functools

import jax
import jax.numpy as jnp
from jax import lax
from jax.experimental import pallas as pl
from jax.experimental.pallas import tpu as pltpu
from jax.experimental.pallas import tpu_sc as plsc

N = 10000
E = 320000
D = 128
NC = 2
NS = 16
NW = NC * NS
RW = 79
EP = NW * RW * 128
CHUNK = 640
NPAD = NS * CHUNK

_f32 = jnp.float32
_mesh = lambda: plsc.VectorSubcoreMesh(core_axis_name="c", subcore_axis_name="s")


def _sc_params():
    cp = pltpu.CompilerParams()
    if "needs_layout_passes" in pltpu.CompilerParams.__dataclass_fields__:
        cp = dataclasses.replace(cp, needs_layout_passes=False)
    return cp


def _fill(ref, value):
    v = jnp.full((16,), value, _f32)

    @pl.loop(0, ref.shape[0], step=16)
    def _(i):
        ref[pl.ds(i, 16)] = v


def _pipe_scatter_ones(idx_v, nrows, acc, ones_v, sems):
    S = len(sems)
    for b in range(S):
        pltpu.async_copy(ones_v, acc.at[idx_v.at[b]], sems[b], add=True)

    @pl.loop(0, nrows - (S - 1), step=S)
    def _(j):
        for b in range(S):
            jj = j + b
            pltpu.make_async_copy(ones_v, acc.at[idx_v.at[jj]],
                                  sems[b]).wait()

            @pl.when(jj + S < nrows)
            def _():
                pltpu.async_copy(ones_v, acc.at[idx_v.at[jj + S]],
                                 sems[b], add=True)

    for b in range(nrows % S):
        pltpu.make_async_copy(ones_v,
                              acc.at[idx_v.at[nrows - nrows % S + b]],
                              sems[b]).wait()


def _reg_gather_row(dst_v, jj, prev_v, out_buf):
    @pl.loop(0, 128, step=16)
    def _(k):
        idx16 = dst_v[jj, pl.ds(k, 16)]
        out_buf[pl.ds(k, 16)] = plsc.load_gather(prev_v, [idx16])


def _pipe_gather_scatter(dst_v, src_v, src_base, prev_v, acc_new, vals,
                         ssems):
    S = len(vals)
    for b in range(S):
        _reg_gather_row(dst_v, b, prev_v, vals[b])
        pltpu.async_copy(vals[b], acc_new.at[src_v.at[src_base + b]],
                         ssems[b], add=True)

    @pl.loop(S, RW - (S - 1), step=S)
    def _(j):
        for b in range(S):
            jj = j + b
            pltpu.make_async_copy(vals[b], acc_new.at[src_v.at[src_base + jj]],
                                  ssems[b]).wait()
            _reg_gather_row(dst_v, jj, prev_v, vals[b])
            pltpu.async_copy(vals[b], acc_new.at[src_v.at[src_base + jj]],
                             ssems[b], add=True)

    for b in range(RW % S):
        jj = RW - RW % S + b
        pltpu.make_async_copy(vals[b], acc_new.at[src_v.at[src_base + jj]],
                              ssems[b]).wait()
        _reg_gather_row(dst_v, jj, prev_v, vals[b])
        pltpu.async_copy(vals[b], acc_new.at[src_v.at[src_base + jj]],
                         ssems[b], add=True)

    for b in range(S):
        jj = ((RW - 1 - b) // S) * S + b
        pltpu.make_async_copy(vals[b], acc_new.at[src_v.at[src_base + jj]],
                              ssems[b]).wait()


def _sc_first(srcr, dstr):
    @functools.partial(
        pl.kernel,
        out_type=[jax.ShapeDtypeStruct((NC, NPAD), _f32),
                  jax.ShapeDtypeStruct((NPAD,), _f32)],
        mesh=_mesh(),
        compiler_params=_sc_params(),
        scratch_types=[
            pltpu.VMEM((2 * RW, 128), jnp.int32),
            pltpu.VMEM((RW, 128), jnp.int32),
            pltpu.VMEM((NPAD,), _f32),
            pltpu.VMEM((128,), _f32),
        ] + [pltpu.VMEM((128,), _f32)] * 8 + [
            pltpu.VMEM((CHUNK,), _f32),
            pltpu.VMEM((CHUNK,), _f32),
            pltpu.VMEM_SHARED((NPAD,), _f32),
            pltpu.VMEM_SHARED((NPAD,), _f32),
        ] + [pltpu.SemaphoreType.DMA] * 9,
    )
    def k(srcr_hbm, dstr_hbm, p2_hbm, u3_hbm,
          src_v, dst_v, prev_v, ones_v, *rest):
        vals = rest[:8]
        tmp_v, tmp2_v, acc3, acc2 = rest[8:12]
        sems = rest[12:]
        gsems, ssems = sems[:8], sems[:8]
        aux_sem = sems[8]
        c = lax.axis_index("c")
        s = lax.axis_index("s")
        sl = pl.ds(s * CHUNK, CHUNK)

        pltpu.sync_copy(srcr_hbm.at[s], src_v)
        dst_dma = pltpu.async_copy(dstr_hbm.at[2 * s + c], dst_v, aux_sem)
        _fill(ones_v, 1.0)
        _fill(tmp_v, 1.0)
        pltpu.sync_copy(tmp_v, acc3.at[sl])
        plsc.subcore_barrier()

        _pipe_scatter_ones(src_v, 2 * RW, acc3, ones_v, gsems)
        dst_dma.wait()
        plsc.subcore_barrier()

        @pl.when(c == 0)
        def _():
            pltpu.sync_copy(acc3.at[sl], tmp_v)
            pltpu.sync_copy(tmp_v, acc2.at[sl])

        @pl.when(c != 0)
        def _():
            _fill(tmp2_v, 0.0)
            pltpu.sync_copy(tmp2_v, acc2.at[sl])

        pltpu.sync_copy(acc3, prev_v)
        plsc.subcore_barrier()
        _pipe_gather_scatter(dst_v, src_v, c * RW, prev_v, acc2, vals,
                             ssems)
        plsc.subcore_barrier()

        pltpu.sync_copy(acc2.at[sl], tmp_v)
        pltpu.sync_copy(tmp_v, p2_hbm.at[c].at[sl])

        @pl.when(c == 0)
        def _():
            pltpu.sync_copy(acc3.at[sl], tmp2_v)
            pltpu.sync_copy(tmp2_v, u3_hbm.at[sl])

    return k(srcr, dstr)


def _sc_second(p_prev, srcr, dstr):
    @functools.partial(
        pl.kernel,
        out_type=jax.ShapeDtypeStruct((NC, NPAD), _f32),
        mesh=_mesh(),
        compiler_params=_sc_params(),
        scratch_types=[
            pltpu.VMEM((RW, 128), jnp.int32),
            pltpu.VMEM((RW, 128), jnp.int32),
            pltpu.VMEM((NPAD,), _f32),
        ] + [pltpu.VMEM((128,), _f32)] * 8 + [
            pltpu.VMEM((CHUNK,), _f32),
            pltpu.VMEM((CHUNK,), _f32),
            pltpu.VMEM_SHARED((NPAD,), _f32),
            pltpu.VMEM_SHARED((NPAD,), _f32),
        ] + [pltpu.SemaphoreType.DMA] * 8,
    )
    def k(p_hbm, srcr_hbm, dstr_hbm, out_hbm,
          src_v, dst_v, prev_v, *rest):
        vals = rest[:8]
        tmp_v, tmp2_v, acc_prev, acc_new = rest[8:12]
        ssems = rest[12:]
        c = lax.axis_index("c")
        s = lax.axis_index("s")
        w = c * NS + s
        sl = pl.ds(s * CHUNK, CHUNK)

        pltpu.sync_copy(srcr_hbm.at[w], src_v)
        pltpu.sync_copy(dstr_hbm.at[w], dst_v)

        pltpu.sync_copy(p_hbm.at[0].at[sl], tmp_v)
        pltpu.sync_copy(p_hbm.at[1].at[sl], tmp2_v)

        @pl.loop(0, CHUNK, step=16)
        def _(i):
            tmp_v[pl.ds(i, 16)] = tmp_v[pl.ds(i, 16)] + tmp2_v[pl.ds(i, 16)]

        pltpu.sync_copy(tmp_v, acc_prev.at[sl])

        @pl.when(c == 0)
        def _():
            pltpu.sync_copy(tmp_v, acc_new.at[sl])

        @pl.when(c != 0)
        def _():
            _fill(tmp2_v, 0.0)
            pltpu.sync_copy(tmp2_v, acc_new.at[sl])

        plsc.subcore_barrier()
        pltpu.sync_copy(acc_prev, prev_v)
        _pipe_gather_scatter(dst_v, src_v, 0, prev_v, acc_new, vals,
                             ssems)
        plsc.subcore_barrier()

        pltpu.sync_copy(acc_new.at[sl], tmp_v)
        pltpu.sync_copy(tmp_v, out_hbm.at[c].at[sl])

    return k(p_prev, srcr, dstr)


def _tc_tail(p1, p2, u3, x, W1, b1, W2, b2, W3, b3):
    def body(p1_ref, p2_ref, u3_ref, x_ref, W1_ref, b1_ref, W2_ref, b2_ref,
             W3_ref, b3_ref, out_ref):
        hi = lax.Precision.HIGHEST
        u1v = p1_ref[0:1, :] + p1_ref[1:2, :]
        r = lax.dot_general(u1v, x_ref[...], (((1,), (0,)), ((), ())),
                            precision=hi, preferred_element_type=_f32)
        s1 = jnp.sum(u1v)
        s2 = jnp.sum(p2_ref[...])
        s3 = jnp.sum(u3_ref[...])
        t = lax.dot_general(r, W1_ref[...], (((1,), (1,)), ((), ())),
                            precision=hi, preferred_element_type=_f32)
        t = t + s1 * b1_ref[...]
        t = lax.dot_general(t, W2_ref[...], (((1,), (1,)), ((), ())),
                            precision=hi, preferred_element_type=_f32)
        t = t + s2 * b2_ref[...]
        t = lax.dot_general(t, W3_ref[...], (((1,), (1,)), ((), ())),
                            precision=hi, preferred_element_type=_f32)
        t = t + s3 * b3_ref[...]
        out_ref[...] = t * (1.0 / N)

    return pl.pallas_call(
        body,
        out_shape=jax.ShapeDtypeStruct((1, D), _f32),
    )(p1, p2, u3, x, W1, b1, W2, b2, W3, b3)


def kernel(x, edge_index, batch, W1, b1, W2, b2, W3, b3):
    pad = jnp.full((EP - E,), N, dtype=jnp.int32)
    srcr = jnp.concatenate([edge_index[0], pad]).reshape(NW, RW, 128)
    dstr = jnp.concatenate([edge_index[1], pad]).reshape(NW, RW, 128)

    p2, u3 = _sc_first(srcr.reshape(NS, 2 * RW, 128), dstr)
    p1 = _sc_second(p2, srcr, dstr)

    return _tc_tail(p1[:, :N], p2[:, :N], u3[:N].reshape(1, N), x,
                    W1, b1.reshape(1, D), W2, b2.reshape(1, D), W3,
                    b3.reshape(1, D))


# speedup vs baseline: 1.0270x; 1.0270x over previous
import dataclasses
import functools

import jax
import jax.numpy as jnp
from jax import lax
from jax.experimental import pallas as pl
from jax.experimental.pallas import tpu as pltpu
from jax.experimental.pallas import tpu_sc as plsc

N = 10000
E = 320000
D = 128
NC = 2
NS = 16
NW = NC * NS
RW = 79
EP = NW * RW * 128
CHUNK = 640
NPAD = NS * CHUNK

_f32 = jnp.float32
_mesh = lambda: plsc.VectorSubcoreMesh(core_axis_name="c", subcore_axis_name="s")


def _sc_params():
    cp = pltpu.CompilerParams()
    if "needs_layout_passes" in pltpu.CompilerParams.__dataclass_fields__:
        cp = dataclasses.replace(cp, needs_layout_passes=False)
    return cp


def _fill(ref, value):
    v = jnp.full((16,), value, _f32)

    @pl.loop(0, ref.shape[0], step=16)
    def _(i):
        ref[pl.ds(i, 16)] = v


def _pipe_scatter_ones(idx_v, nrows, acc, ones_v, sems):
    for b in range(4):
        pltpu.async_copy(ones_v, acc.at[idx_v.at[b]], sems[b], add=True)

    @pl.loop(0, nrows - 3, step=4)
    def _(j):
        for b in range(4):
            jj = j + b
            pltpu.make_async_copy(ones_v, acc.at[idx_v.at[jj]],
                                  sems[b]).wait()

            @pl.when(jj + 4 < nrows)
            def _():
                pltpu.async_copy(ones_v, acc.at[idx_v.at[jj + 4]],
                                 sems[b], add=True)

    for b in range(nrows % 4):
        pltpu.make_async_copy(ones_v,
                              acc.at[idx_v.at[nrows - nrows % 4 + b]],
                              sems[b]).wait()


def _reg_gather_row(dst_v, jj, prev_v, out_buf):
    @pl.loop(0, 128, step=16)
    def _(k):
        idx16 = dst_v[jj, pl.ds(k, 16)]
        out_buf[pl.ds(k, 16)] = plsc.load_gather(prev_v, [idx16])


def _pipe_gather_scatter(dst_v, src_v, src_base, prev_v, acc_new, vals,
                         ssems):
    for b in range(4):
        _reg_gather_row(dst_v, b, prev_v, vals[b])
        pltpu.async_copy(vals[b], acc_new.at[src_v.at[src_base + b]],
                         ssems[b], add=True)

    @pl.loop(4, RW - 3, step=4)
    def _(j):
        for b in range(4):
            jj = j + b
            pltpu.make_async_copy(vals[b], acc_new.at[src_v.at[src_base + jj]],
                                  ssems[b]).wait()
            _reg_gather_row(dst_v, jj, prev_v, vals[b])
            pltpu.async_copy(vals[b], acc_new.at[src_v.at[src_base + jj]],
                             ssems[b], add=True)

    for b in range(RW % 4):
        jj = RW - RW % 4 + b
        pltpu.make_async_copy(vals[b], acc_new.at[src_v.at[src_base + jj]],
                              ssems[b]).wait()
        _reg_gather_row(dst_v, jj, prev_v, vals[b])
        pltpu.async_copy(vals[b], acc_new.at[src_v.at[src_base + jj]],
                         ssems[b], add=True)

    for b in range(4):
        jj = ((RW - 1 - b) // 4) * 4 + b
        pltpu.make_async_copy(vals[b], acc_new.at[src_v.at[src_base + jj]],
                              ssems[b]).wait()


def _sc_first(srcr, dstr):
    @functools.partial(
        pl.kernel,
        out_type=[jax.ShapeDtypeStruct((NC, NPAD), _f32),
                  jax.ShapeDtypeStruct((NPAD,), _f32)],
        mesh=_mesh(),
        compiler_params=_sc_params(),
        scratch_types=[
            pltpu.VMEM((2 * RW, 128), jnp.int32),
            pltpu.VMEM((RW, 128), jnp.int32),
            pltpu.VMEM((NPAD,), _f32),
            pltpu.VMEM((128,), _f32),
            pltpu.VMEM((128,), _f32),
            pltpu.VMEM((128,), _f32),
            pltpu.VMEM((128,), _f32),
            pltpu.VMEM((128,), _f32),
            pltpu.VMEM((CHUNK,), _f32),
            pltpu.VMEM((CHUNK,), _f32),
            pltpu.VMEM_SHARED((NPAD,), _f32),
            pltpu.VMEM_SHARED((NPAD,), _f32),
            pltpu.SemaphoreType.DMA,
            pltpu.SemaphoreType.DMA,
            pltpu.SemaphoreType.DMA,
            pltpu.SemaphoreType.DMA,
            pltpu.SemaphoreType.DMA,
            pltpu.SemaphoreType.DMA,
            pltpu.SemaphoreType.DMA,
            pltpu.SemaphoreType.DMA,
        ],
    )
    def k(srcr_hbm, dstr_hbm, p2_hbm, u3_hbm,
          src_v, dst_v, prev_v, ones_v, v0, v1, v2, v3, tmp_v, tmp2_v,
          acc3, acc2, *sems):
        vals = (v0, v1, v2, v3)
        gsems, ssems = sems[:4], sems[4:]
        c = lax.axis_index("c")
        s = lax.axis_index("s")
        sl = pl.ds(s * CHUNK, CHUNK)

        pltpu.sync_copy(srcr_hbm.at[s], src_v)
        dst_dma = pltpu.async_copy(dstr_hbm.at[2 * s + c], dst_v, ssems[0])
        _fill(ones_v, 1.0)
        _fill(tmp_v, 1.0)
        pltpu.sync_copy(tmp_v, acc3.at[sl])
        plsc.subcore_barrier()

        _pipe_scatter_ones(src_v, 2 * RW, acc3, ones_v, gsems)
        dst_dma.wait()
        plsc.subcore_barrier()

        prev_dma = pltpu.async_copy(acc3, prev_v, gsems[1])

        @pl.when(c == 0)
        def _():
            pltpu.sync_copy(acc3.at[sl], tmp_v)
            pltpu.sync_copy(tmp_v, acc2.at[sl])

        @pl.when(c != 0)
        def _():
            _fill(tmp2_v, 0.0)
            pltpu.sync_copy(tmp2_v, acc2.at[sl])

        prev_dma.wait()
        plsc.subcore_barrier()
        _pipe_gather_scatter(dst_v, src_v, c * RW, prev_v, acc2, vals,
                             ssems)
        plsc.subcore_barrier()

        pltpu.sync_copy(acc2.at[sl], tmp_v)
        pltpu.sync_copy(tmp_v, p2_hbm.at[c].at[sl])

        @pl.when(c == 0)
        def _():
            pltpu.sync_copy(acc3.at[sl], tmp2_v)
            pltpu.sync_copy(tmp2_v, u3_hbm.at[sl])

    return k(srcr, dstr)


def _sc_second(p_prev, srcr, dstr):
    @functools.partial(
        pl.kernel,
        out_type=jax.ShapeDtypeStruct((NC, NPAD), _f32),
        mesh=_mesh(),
        compiler_params=_sc_params(),
        scratch_types=[
            pltpu.VMEM((RW, 128), jnp.int32),
            pltpu.VMEM((RW, 128), jnp.int32),
            pltpu.VMEM((NPAD,), _f32),
            pltpu.VMEM((128,), _f32),
            pltpu.VMEM((128,), _f32),
            pltpu.VMEM((128,), _f32),
            pltpu.VMEM((128,), _f32),
            pltpu.VMEM((CHUNK,), _f32),
            pltpu.VMEM((CHUNK,), _f32),
            pltpu.VMEM_SHARED((NPAD,), _f32),
            pltpu.VMEM_SHARED((NPAD,), _f32),
            pltpu.SemaphoreType.DMA,
            pltpu.SemaphoreType.DMA,
            pltpu.SemaphoreType.DMA,
            pltpu.SemaphoreType.DMA,
            pltpu.SemaphoreType.DMA,
            pltpu.SemaphoreType.DMA,
            pltpu.SemaphoreType.DMA,
            pltpu.SemaphoreType.DMA,
        ],
    )
    def k(p_hbm, srcr_hbm, dstr_hbm, out_hbm,
          src_v, dst_v, prev_v, v0, v1, v2, v3, tmp_v, tmp2_v, acc_prev,
          acc_new, *sems):
        vals = (v0, v1, v2, v3)
        gsems, ssems = sems[:4], sems[4:]
        c = lax.axis_index("c")
        s = lax.axis_index("s")
        w = c * NS + s
        sl = pl.ds(s * CHUNK, CHUNK)

        src_dma = pltpu.async_copy(srcr_hbm.at[w], src_v, ssems[0])
        dst_dma = pltpu.async_copy(dstr_hbm.at[w], dst_v, ssems[1])

        pltpu.sync_copy(p_hbm.at[0].at[sl], tmp_v)
        pltpu.sync_copy(p_hbm.at[1].at[sl], tmp2_v)

        @pl.loop(0, CHUNK, step=16)
        def _(i):
            tmp_v[pl.ds(i, 16)] = tmp_v[pl.ds(i, 16)] + tmp2_v[pl.ds(i, 16)]

        pltpu.sync_copy(tmp_v, acc_prev.at[sl])

        @pl.when(c == 0)
        def _():
            pltpu.sync_copy(tmp_v, acc_new.at[sl])

        @pl.when(c != 0)
        def _():
            _fill(tmp2_v, 0.0)
            pltpu.sync_copy(tmp2_v, acc_new.at[sl])

        src_dma.wait()
        dst_dma.wait()
        plsc.subcore_barrier()
        pltpu.sync_copy(acc_prev, prev_v)
        _pipe_gather_scatter(dst_v, src_v, 0, prev_v, acc_new, vals,
                             ssems)
        plsc.subcore_barrier()

        pltpu.sync_copy(acc_new.at[sl], tmp_v)
        pltpu.sync_copy(tmp_v, out_hbm.at[c].at[sl])

    return k(p_prev, srcr, dstr)


def _tc_tail(p1, p2, u3, x, W1, b1, W2, b2, W3, b3):
    def body(p1_ref, p2_ref, u3_ref, x_ref, W1_ref, b1_ref, W2_ref, b2_ref,
             W3_ref, b3_ref, out_ref):
        hi = lax.Precision.HIGHEST
        u1v = p1_ref[0:1, :] + p1_ref[1:2, :]
        r = lax.dot_general(u1v, x_ref[...], (((1,), (0,)), ((), ())),
                            precision=hi, preferred_element_type=_f32)
        s1 = jnp.sum(u1v)
        s2 = jnp.sum(p2_ref[...])
        s3 = jnp.sum(u3_ref[...])
        t = lax.dot_general(r, W1_ref[...], (((1,), (1,)), ((), ())),
                            precision=hi, preferred_element_type=_f32)
        t = t + s1 * b1_ref[...]
        t = lax.dot_general(t, W2_ref[...], (((1,), (1,)), ((), ())),
                            precision=hi, preferred_element_type=_f32)
        t = t + s2 * b2_ref[...]
        t = lax.dot_general(t, W3_ref[...], (((1,), (1,)), ((), ())),
                            precision=hi, preferred_element_type=_f32)
        t = t + s3 * b3_ref[...]
        out_ref[...] = t * (1.0 / N)

    return pl.pallas_call(
        body,
        out_shape=jax.ShapeDtypeStruct((1, D), _f32),
    )(p1, p2, u3, x, W1, b1, W2, b2, W3, b3)


def kernel(x, edge_index, batch, W1, b1, W2, b2, W3, b3):
    pad = jnp.full((EP - E,), N, dtype=jnp.int32)
    srcr = jnp.concatenate([edge_index[0], pad]).reshape(NW, RW, 128)
    dstr = jnp.concatenate([edge_index[1], pad]).reshape(NW, RW, 128)

    p2, u3 = _sc_first(srcr.reshape(NS, 2 * RW, 128), dstr)
    p1 = _sc_second(p2, srcr, dstr)

    return _tc_tail(p1[:, :N], p2[:, :N], u3[:N].reshape(1, N), x,
                    W1, b1.reshape(1, D), W2, b2.reshape(1, D), W3,
                    b3.reshape(1, D))


# speedup vs baseline: 1.0380x; 1.0107x over previous
import dataclasses
import functools

import jax
import jax.numpy as jnp
from jax import lax
from jax.experimental import pallas as pl
from jax.experimental.pallas import tpu as pltpu
from jax.experimental.pallas import tpu_sc as plsc

N = 10000
E = 320000
D = 128
NC = 2
NS = 16
NW = NC * NS
RW = 79
EP = NW * RW * 128
CHUNK = 640
NPAD = NS * CHUNK

_f32 = jnp.float32
_mesh = lambda: plsc.VectorSubcoreMesh(core_axis_name="c", subcore_axis_name="s")


def _sc_params():
    cp = pltpu.CompilerParams()
    if "needs_layout_passes" in pltpu.CompilerParams.__dataclass_fields__:
        cp = dataclasses.replace(cp, needs_layout_passes=False)
    return cp


def _fill(ref, value):
    v = jnp.full((16,), value, _f32)

    @pl.loop(0, ref.shape[0], step=16)
    def _(i):
        ref[pl.ds(i, 16)] = v


def _pipe_scatter_ones(idx_v, nrows, acc, ones_v, sems):
    for b in range(4):
        pltpu.async_copy(ones_v, acc.at[idx_v.at[b]], sems[b], add=True)

    @pl.loop(0, nrows - 3, step=4)
    def _(j):
        for b in range(4):
            jj = j + b
            pltpu.make_async_copy(ones_v, acc.at[idx_v.at[jj]],
                                  sems[b]).wait()

            @pl.when(jj + 4 < nrows)
            def _():
                pltpu.async_copy(ones_v, acc.at[idx_v.at[jj + 4]],
                                 sems[b], add=True)

    for b in range(nrows % 4):
        pltpu.make_async_copy(ones_v,
                              acc.at[idx_v.at[nrows - nrows % 4 + b]],
                              sems[b]).wait()


def _reg_gather_row(dst_v, jj, prev_v, out_buf):
    @pl.loop(0, 128, step=16)
    def _(k):
        idx16 = dst_v[jj, pl.ds(k, 16)]
        out_buf[pl.ds(k, 16)] = plsc.load_gather(prev_v, [idx16])


def _pipe_gather_scatter(dst_v, src_v, src_base, prev_v, acc_new, vals,
                         ssems):
    for b in range(4):
        _reg_gather_row(dst_v, b, prev_v, vals[b])
        pltpu.async_copy(vals[b], acc_new.at[src_v.at[src_base + b]],
                         ssems[b], add=True)

    @pl.loop(4, RW - 3, step=4)
    def _(j):
        for b in range(4):
            jj = j + b
            pltpu.make_async_copy(vals[b], acc_new.at[src_v.at[src_base + jj]],
                                  ssems[b]).wait()
            _reg_gather_row(dst_v, jj, prev_v, vals[b])
            pltpu.async_copy(vals[b], acc_new.at[src_v.at[src_base + jj]],
                             ssems[b], add=True)

    for b in range(RW % 4):
        jj = RW - RW % 4 + b
        pltpu.make_async_copy(vals[b], acc_new.at[src_v.at[src_base + jj]],
                              ssems[b]).wait()
        _reg_gather_row(dst_v, jj, prev_v, vals[b])
        pltpu.async_copy(vals[b], acc_new.at[src_v.at[src_base + jj]],
                         ssems[b], add=True)

    for b in range(4):
        jj = ((RW - 1 - b) // 4) * 4 + b
        pltpu.make_async_copy(vals[b], acc_new.at[src_v.at[src_base + jj]],
                              ssems[b]).wait()


def _sc_first(srcr, dstr):
    @functools.partial(
        pl.kernel,
        out_type=[jax.ShapeDtypeStruct((NC, NPAD), _f32),
                  jax.ShapeDtypeStruct((NPAD,), _f32)],
        mesh=_mesh(),
        compiler_params=_sc_params(),
        scratch_types=[
            pltpu.VMEM((2 * RW, 128), jnp.int32),
            pltpu.VMEM((RW, 128), jnp.int32),
            pltpu.VMEM((NPAD,), _f32),
            pltpu.VMEM((128,), _f32),
            pltpu.VMEM((128,), _f32),
            pltpu.VMEM((128,), _f32),
            pltpu.VMEM((128,), _f32),
            pltpu.VMEM((128,), _f32),
            pltpu.VMEM((CHUNK,), _f32),
            pltpu.VMEM((CHUNK,), _f32),
            pltpu.VMEM_SHARED((NPAD,), _f32),
            pltpu.VMEM_SHARED((NPAD,), _f32),
            pltpu.SemaphoreType.DMA,
            pltpu.SemaphoreType.DMA,
            pltpu.SemaphoreType.DMA,
            pltpu.SemaphoreType.DMA,
            pltpu.SemaphoreType.DMA,
            pltpu.SemaphoreType.DMA,
            pltpu.SemaphoreType.DMA,
            pltpu.SemaphoreType.DMA,
        ],
    )
    def k(srcr_hbm, dstr_hbm, p2_hbm, u3_hbm,
          src_v, dst_v, prev_v, ones_v, v0, v1, v2, v3, tmp_v, tmp2_v,
          acc3, acc2, *sems):
        vals = (v0, v1, v2, v3)
        gsems, ssems = sems[:4], sems[4:]
        c = lax.axis_index("c")
        s = lax.axis_index("s")
        sl = pl.ds(s * CHUNK, CHUNK)

        pltpu.sync_copy(srcr_hbm.at[s], src_v)
        dst_dma = pltpu.async_copy(dstr_hbm.at[2 * s + c], dst_v, ssems[0])
        _fill(ones_v, 1.0)
        _fill(tmp_v, 1.0)
        pltpu.sync_copy(tmp_v, acc3.at[sl])
        plsc.subcore_barrier()

        _pipe_scatter_ones(src_v, 2 * RW, acc3, ones_v, gsems)
        dst_dma.wait()
        plsc.subcore_barrier()

        prev_dma = pltpu.async_copy(acc3, prev_v, gsems[1])

        @pl.when(c == 0)
        def _():
            pltpu.sync_copy(acc3.at[sl], tmp_v)
            pltpu.sync_copy(tmp_v, acc2.at[sl])

        @pl.when(c != 0)
        def _():
            _fill(tmp2_v, 0.0)
            pltpu.sync_copy(tmp2_v, acc2.at[sl])

        prev_dma.wait()
        plsc.subcore_barrier()
        _pipe_gather_scatter(dst_v, src_v, c * RW, prev_v, acc2, vals,
                             ssems)
        plsc.subcore_barrier()

        pltpu.sync_copy(acc2.at[sl], tmp_v)
        p2_dma = pltpu.async_copy(tmp_v, p2_hbm.at[c].at[sl], gsems[2])

        @pl.when(c == 0)
        def _():
            pltpu.sync_copy(acc3.at[sl], tmp2_v)
            pltpu.sync_copy(tmp2_v, u3_hbm.at[sl])

        p2_dma.wait()

    return k(srcr, dstr)


def _sc_second(p_prev, srcr, dstr):
    @functools.partial(
        pl.kernel,
        out_type=jax.ShapeDtypeStruct((NC, NPAD), _f32),
        mesh=_mesh(),
        compiler_params=_sc_params(),
        scratch_types=[
            pltpu.VMEM((RW, 128), jnp.int32),
            pltpu.VMEM((RW, 128), jnp.int32),
            pltpu.VMEM((NPAD,), _f32),
            pltpu.VMEM((128,), _f32),
            pltpu.VMEM((128,), _f32),
            pltpu.VMEM((128,), _f32),
            pltpu.VMEM((128,), _f32),
            pltpu.VMEM((CHUNK,), _f32),
            pltpu.VMEM((CHUNK,), _f32),
            pltpu.VMEM_SHARED((NPAD,), _f32),
            pltpu.VMEM_SHARED((NPAD,), _f32),
            pltpu.SemaphoreType.DMA,
            pltpu.SemaphoreType.DMA,
            pltpu.SemaphoreType.DMA,
            pltpu.SemaphoreType.DMA,
            pltpu.SemaphoreType.DMA,
            pltpu.SemaphoreType.DMA,
            pltpu.SemaphoreType.DMA,
            pltpu.SemaphoreType.DMA,
        ],
    )
    def k(p_hbm, srcr_hbm, dstr_hbm, out_hbm,
          src_v, dst_v, prev_v, v0, v1, v2, v3, tmp_v, tmp2_v, acc_prev,
          acc_new, *sems):
        vals = (v0, v1, v2, v3)
        gsems, ssems = sems[:4], sems[4:]
        c = lax.axis_index("c")
        s = lax.axis_index("s")
        w = c * NS + s
        sl = pl.ds(s * CHUNK, CHUNK)

        src_dma = pltpu.async_copy(srcr_hbm.at[w], src_v, ssems[0])
        dst_dma = pltpu.async_copy(dstr_hbm.at[w], dst_v, ssems[1])

        pa_dma = pltpu.async_copy(p_hbm.at[0].at[sl], tmp_v, ssems[2])
        pb_dma = pltpu.async_copy(p_hbm.at[1].at[sl], tmp2_v, ssems[3])
        pa_dma.wait()
        pb_dma.wait()

        @pl.loop(0, CHUNK, step=16)
        def _(i):
            tmp_v[pl.ds(i, 16)] = tmp_v[pl.ds(i, 16)] + tmp2_v[pl.ds(i, 16)]

        pltpu.sync_copy(tmp_v, acc_prev.at[sl])

        @pl.when(c == 0)
        def _():
            pltpu.sync_copy(tmp_v, acc_new.at[sl])

        @pl.when(c != 0)
        def _():
            _fill(tmp2_v, 0.0)
            pltpu.sync_copy(tmp2_v, acc_new.at[sl])

        src_dma.wait()
        dst_dma.wait()
        plsc.subcore_barrier()
        pltpu.sync_copy(acc_prev, prev_v)
        _pipe_gather_scatter(dst_v, src_v, 0, prev_v, acc_new, vals,
                             ssems)
        plsc.subcore_barrier()

        pltpu.sync_copy(acc_new.at[sl], tmp_v)
        pltpu.sync_copy(tmp_v, out_hbm.at[c].at[sl])

    return k(p_prev, srcr, dstr)


def _tc_tail(p1, p2, u3, x, W1, b1, W2, b2, W3, b3):
    def body(p1_ref, p2_ref, u3_ref, x_ref, W1_ref, b1_ref, W2_ref, b2_ref,
             W3_ref, b3_ref, out_ref):
        hi = lax.Precision.HIGHEST
        u1v = p1_ref[0:1, :] + p1_ref[1:2, :]
        r = lax.dot_general(u1v, x_ref[...], (((1,), (0,)), ((), ())),
                            precision=hi, preferred_element_type=_f32)
        s1 = jnp.sum(u1v)
        s2 = jnp.sum(p2_ref[...])
        s3 = jnp.sum(u3_ref[...])
        t = lax.dot_general(r, W1_ref[...], (((1,), (1,)), ((), ())),
                            precision=hi, preferred_element_type=_f32)
        t = t + s1 * b1_ref[...]
        t = lax.dot_general(t, W2_ref[...], (((1,), (1,)), ((), ())),
                            precision=hi, preferred_element_type=_f32)
        t = t + s2 * b2_ref[...]
        t = lax.dot_general(t, W3_ref[...], (((1,), (1,)), ((), ())),
                            precision=hi, preferred_element_type=_f32)
        t = t + s3 * b3_ref[...]
        out_ref[...] = t * (1.0 / N)

    return pl.pallas_call(
        body,
        out_shape=jax.ShapeDtypeStruct((1, D), _f32),
    )(p1, p2, u3, x, W1, b1, W2, b2, W3, b3)


def kernel(x, edge_index, batch, W1, b1, W2, b2, W3, b3):
    pad = jnp.full((EP - E,), N, dtype=jnp.int32)
    srcr = jnp.concatenate([edge_index[0], pad]).reshape(NW, RW, 128)
    dstr = jnp.concatenate([edge_index[1], pad]).reshape(NW, RW, 128)

    p2, u3 = _sc_first(srcr.reshape(NS, 2 * RW, 128), dstr)
    p1 = _sc_second(p2, srcr, dstr)

    return _tc_tail(p1[:, :N], p2[:, :N], u3[:N].reshape(1, N), x,
                    W1, b1.reshape(1, D), W2, b2.reshape(1, D), W3,
                    b3.reshape(1, D))
